# Initial kernel scaffold; baseline (speedup 1.0000x reference)
import jax
import jax.numpy as jnp
from jax import lax
from jax.experimental import pallas as pl
from jax.experimental.pallas import tpu as pltpu
from jax.experimental.pallas import tpu_sc as plsc

B = 16384
HIST = 200
D = 32
NC = 2
NS = 16
NW = NC * NS
RPW = B // NW
CHUNK = 64
NCHUNK = RPW // CHUNK
GA = 128
GB = HIST - GA


def _sc_body(idx_hbm, emb_hbm, out_hbm, idx_v, rows_a, rows_b, emb0_v, out_v, sem):
    wid = lax.axis_index("s") * NC + lax.axis_index("c")
    row0 = wid * RPW
    pltpu.sync_copy(emb_hbm.at[0], emb0_v)
    e0_lo = emb0_v[pl.ds(0, 16)]
    e0_hi = emb0_v[pl.ds(16, 16)]
    lane = lax.iota(jnp.int32, 16)
    lane_mask = lane < (HIST - 12 * 16)
    zero_v = jnp.zeros((16,), jnp.float32)
    one_v = jnp.ones((16,), jnp.float32)
    hist_v = jnp.full((16,), float(HIST), jnp.float32)

    def chunk_body(c, _):
        base = row0 + c * CHUNK
        pltpu.sync_copy(idx_hbm.at[pl.ds(base * HIST, CHUNK * HIST)],
                        idx_v.at[pl.ds(0, CHUNK * HIST)])

        def row_body(i, _):
            off = i * HIST
            cp_a = pltpu.async_copy(emb_hbm.at[idx_v.at[pl.ds(off, GA)]], rows_a, sem)
            cp_b = pltpu.async_copy(emb_hbm.at[idx_v.at[pl.ds(off + GA, GB)]], rows_b, sem)
            cnt = zero_v
            for k in range(13):
                v = idx_v[pl.ds(off + k * 16, 16)]
                iszero = v == 0
                if k == 12:
                    iszero = jnp.logical_and(iszero, lane_mask)
                cnt = cnt + jnp.where(iszero, one_v, zero_v)
            nz = jnp.sum(cnt)
            nzv = jnp.full((16,), nz, jnp.float32)
            cp_a.wait()

            def acc_a(j, carry):
                lo, hi = carry
                return lo + rows_a[j, pl.ds(0, 16)], hi + rows_a[j, pl.ds(16, 16)]

            lo, hi = lax.fori_loop(0, GA, acc_a, (zero_v, zero_v))
            cp_b.wait()

            def acc_b(j, carry):
                lo, hi = carry
                return lo + rows_b[j, pl.ds(0, 16)], hi + rows_b[j, pl.ds(16, 16)]

            lo, hi = lax.fori_loop(0, GB, acc_b, (lo, hi))
            denom = (hist_v - nzv) + 1e-8
            out_v[i, pl.ds(0, 16)] = (lo - nzv * e0_lo) / denom
            out_v[i, pl.ds(16, 16)] = (hi - nzv * e0_hi) / denom
            return 0

        lax.fori_loop(0, CHUNK, row_body, 0)
        pltpu.sync_copy(out_v, out_hbm.at[pl.ds(base, CHUNK)])
        return 0

    lax.fori_loop(0, NCHUNK, chunk_body, 0)


def kernel(inputs, embeddings):
    idx_flat = inputs.reshape(-1)
    mesh = plsc.VectorSubcoreMesh(core_axis_name="c", subcore_axis_name="s",
                                  num_cores=NC, num_subcores=NS)
    f = pl.kernel(
        _sc_body,
        out_type=jax.ShapeDtypeStruct((B, D), jnp.float32),
        mesh=mesh,
        scratch_types=[
            pltpu.VMEM((CHUNK * HIST + 16,), jnp.int32),
            pltpu.VMEM((GA, D), jnp.float32),
            pltpu.VMEM((GB, D), jnp.float32),
            pltpu.VMEM((D,), jnp.float32),
            pltpu.VMEM((CHUNK, D), jnp.float32),
            pltpu.SemaphoreType.DMA,
        ],
    )
    return f(idx_flat, embeddings)


# speedup vs baseline: 9.0199x; 9.0199x over previous
import jax
import jax.numpy as jnp
from jax import lax
from jax.experimental import pallas as pl
from jax.experimental.pallas import tpu as pltpu
from jax.experimental.pallas import tpu_sc as plsc

B = 16384
HIST = 200
D = 32
NC = 2
NS = 16
NW = NC * NS
RPW = B // NW
CHUNK = 64
NCHUNK = RPW // CHUNK
GA = 128
GB = HIST - GA


def _sc_body(idx_hbm, emb_hbm, out_hbm, idx_v, rows_a, rows_b, emb0_v, out_v, sem):
    wid = lax.axis_index("s") * NC + lax.axis_index("c")
    row0 = wid * RPW
    pltpu.sync_copy(emb_hbm.at[0], emb0_v)
    e0_lo = emb0_v[pl.ds(0, 16)]
    e0_hi = emb0_v[pl.ds(16, 16)]
    lane = lax.iota(jnp.int32, 16)
    lane_mask = lane < (HIST - 12 * 16)
    zero_v = jnp.zeros((16,), jnp.float32)
    one_v = jnp.ones((16,), jnp.float32)
    hist_v = jnp.full((16,), float(HIST), jnp.float32)

    def chunk_body(c, _):
        base = row0 + c * CHUNK
        pltpu.sync_copy(idx_hbm.at[pl.ds(base * HIST, CHUNK * HIST)],
                        idx_v.at[pl.ds(0, CHUNK * HIST)])

        def row_body(i, _):
            off = i * HIST
            cp_a = pltpu.async_copy(emb_hbm.at[idx_v.at[pl.ds(off, GA)]], rows_a, sem)
            cp_b = pltpu.async_copy(emb_hbm.at[idx_v.at[pl.ds(off + GA, GB)]], rows_b, sem)
            cnt = jnp.zeros((16,), jnp.int32)
            for k in range(13):
                v = idx_v[pl.ds(off + k * 16, 16)]
                iszero = v == 0
                if k == 12:
                    iszero = jnp.logical_and(iszero, lane_mask)
                cnt = cnt + plsc.all_reduce_population_count(iszero)
            nzv = cnt.astype(jnp.float32)
            cp_a.wait()

            def acc_a(j, carry):
                lo, hi = carry
                return lo + rows_a[j, pl.ds(0, 16)], hi + rows_a[j, pl.ds(16, 16)]

            lo, hi = lax.fori_loop(0, GA, acc_a, (zero_v, zero_v))
            cp_b.wait()

            def acc_b(j, carry):
                lo, hi = carry
                return lo + rows_b[j, pl.ds(0, 16)], hi + rows_b[j, pl.ds(16, 16)]

            lo, hi = lax.fori_loop(0, GB, acc_b, (lo, hi))
            denom = (hist_v - nzv) + 1e-8
            out_v[i, pl.ds(0, 16)] = (lo - nzv * e0_lo) / denom
            out_v[i, pl.ds(16, 16)] = (hi - nzv * e0_hi) / denom
            return 0

        lax.fori_loop(0, CHUNK, row_body, 0)
        pltpu.sync_copy(out_v, out_hbm.at[pl.ds(base, CHUNK)])
        return 0

    lax.fori_loop(0, NCHUNK, chunk_body, 0)


def kernel(inputs, embeddings):
    idx_flat = inputs.reshape(-1)
    mesh = plsc.VectorSubcoreMesh(core_axis_name="c", subcore_axis_name="s",
                                  num_cores=NC, num_subcores=NS)
    f = pl.kernel(
        _sc_body,
        out_type=jax.ShapeDtypeStruct((B, D), jnp.float32),
        mesh=mesh,
        compiler_params=pltpu.CompilerParams(needs_layout_passes=False,
                                             use_tc_tiling_on_sc=False),
        scratch_types=[
            pltpu.VMEM((CHUNK * HIST + 16,), jnp.int32),
            pltpu.VMEM((GA, D), jnp.float32),
            pltpu.VMEM((GB, D), jnp.float32),
            pltpu.VMEM((D,), jnp.float32),
            pltpu.VMEM((CHUNK, D), jnp.float32),
            pltpu.SemaphoreType.DMA,
        ],
    )
    return f(idx_flat, embeddings)


# speedup vs baseline: 16.3121x; 1.8085x over previous
import jax
import jax.numpy as jnp
from jax import lax
from jax.experimental import pallas as pl
from jax.experimental.pallas import tpu as pltpu
from jax.experimental.pallas import tpu_sc as plsc

B = 16384
HIST = 200
D = 32
NC = 2
NS = 16
NW = NC * NS
RPW = B // NW
CHUNK = 64
NCHUNK = RPW // CHUNK
GA = 128
GB = HIST - GA
NBUF = 4
NSLICE = -(-HIST // 16)


def _sc_body(idx_hbm, emb_hbm, out_hbm, idx_v, rows_a, rows_b, emb0_v, out_v, sems):
    wid = lax.axis_index("s") * NC + lax.axis_index("c")
    row0 = wid * RPW
    pltpu.sync_copy(emb_hbm.at[0], emb0_v)
    e0_lo = emb0_v[pl.ds(0, 16)]
    e0_hi = emb0_v[pl.ds(16, 16)]
    lane = lax.iota(jnp.int32, 16)
    lane_mask = lane < (HIST - (NSLICE - 1) * 16)
    zero_v = jnp.zeros((16,), jnp.float32)
    hist_v = jnp.full((16,), float(HIST), jnp.float32)

    def issue(r, s):
        off = r * HIST
        pltpu.async_copy(emb_hbm.at[idx_v.at[pl.ds(off, GA)]],
                         rows_a.at[pl.ds(s * GA, GA)], sems.at[s])
        pltpu.async_copy(emb_hbm.at[idx_v.at[pl.ds(off + GA, GB)]],
                         rows_b.at[pl.ds(s * GB, GB)], sems.at[s])

    def drain(s):
        pltpu.make_async_copy(emb_hbm.at[pl.ds(0, GA)],
                              rows_a.at[pl.ds(s * GA, GA)], sems.at[s]).wait()
        pltpu.make_async_copy(emb_hbm.at[pl.ds(0, GB)],
                              rows_b.at[pl.ds(s * GB, GB)], sems.at[s]).wait()

    def chunk_body(c, _):
        base = row0 + c * CHUNK
        pltpu.sync_copy(idx_hbm.at[pl.ds(base * HIST, CHUNK * HIST)],
                        idx_v.at[pl.ds(0, CHUNK * HIST)])
        for s in range(NBUF - 1):
            issue(s, s)

        def block_body(q, _):
            r0 = q * NBUF
            for s in range(NBUF):
                r = r0 + s
                nxt = r + (NBUF - 1)

                @pl.when(nxt < CHUNK)
                def _():
                    issue(nxt, (s + NBUF - 1) % NBUF)

                off = r * HIST
                cnt = jnp.zeros((16,), jnp.int32)
                for k in range(NSLICE):
                    v = idx_v[pl.ds(off + k * 16, 16)]
                    iszero = v == 0
                    if k == NSLICE - 1:
                        iszero = jnp.logical_and(iszero, lane_mask)
                    cnt = cnt + plsc.all_reduce_population_count(iszero)
                nzv = cnt.astype(jnp.float32)

                drain(s)

                def acc_a(t, carry):
                    lo, hi = carry
                    j0 = s * GA + t * 8
                    for u in range(8):
                        lo = lo + rows_a[j0 + u, pl.ds(0, 16)]
                        hi = hi + rows_a[j0 + u, pl.ds(16, 16)]
                    return lo, hi

                lo, hi = lax.fori_loop(0, GA // 8, acc_a, (zero_v, zero_v))

                def acc_b(t, carry):
                    lo, hi = carry
                    j0 = s * GB + t * 8
                    for u in range(8):
                        lo = lo + rows_b[j0 + u, pl.ds(0, 16)]
                        hi = hi + rows_b[j0 + u, pl.ds(16, 16)]
                    return lo, hi

                lo, hi = lax.fori_loop(0, GB // 8, acc_b, (lo, hi))
                denom = (hist_v - nzv) + 1e-8
                out_v[r, pl.ds(0, 16)] = (lo - nzv * e0_lo) / denom
                out_v[r, pl.ds(16, 16)] = (hi - nzv * e0_hi) / denom
            return 0

        lax.fori_loop(0, CHUNK // NBUF, block_body, 0)
        pltpu.sync_copy(out_v, out_hbm.at[pl.ds(base, CHUNK)])
        return 0

    lax.fori_loop(0, NCHUNK, chunk_body, 0)


def kernel(inputs, embeddings):
    idx_flat = inputs.reshape(-1)
    mesh = plsc.VectorSubcoreMesh(core_axis_name="c", subcore_axis_name="s",
                                  num_cores=NC, num_subcores=NS)
    f = pl.kernel(
        _sc_body,
        out_type=jax.ShapeDtypeStruct((B, D), jnp.float32),
        mesh=mesh,
        compiler_params=pltpu.CompilerParams(needs_layout_passes=False,
                                             use_tc_tiling_on_sc=False),
        scratch_types=[
            pltpu.VMEM((CHUNK * HIST + 16,), jnp.int32),
            pltpu.VMEM((NBUF * GA, D), jnp.float32),
            pltpu.VMEM((NBUF * GB, D), jnp.float32),
            pltpu.VMEM((D,), jnp.float32),
            pltpu.VMEM((CHUNK, D), jnp.float32),
            pltpu.SemaphoreType.DMA((NBUF,)),
        ],
    )
    return f(idx_flat, embeddings)


# speedup vs baseline: 16.3416x; 1.0018x over previous
import jax
import jax.numpy as jnp
from jax import lax
from jax.experimental import pallas as pl
from jax.experimental.pallas import tpu as pltpu
from jax.experimental.pallas import tpu_sc as plsc

B = 16384
HIST = 200
D = 32
NC = 2
NS = 16
NW = NC * NS
RPW = B // NW
CHUNK = 64
NCHUNK = RPW // CHUNK
GA = 128
GB = HIST - GA
NBUF = 4


def _sc_body(idx_hbm, emb_hbm, out_hbm, idx_v, rows_a, rows_b, emb0_v, out_v, sems):
    wid = lax.axis_index("s") * NC + lax.axis_index("c")
    row0 = wid * RPW
    pltpu.sync_copy(emb_hbm.at[0], emb0_v)
    e0_lo = emb0_v[pl.ds(0, 16)]
    e0_hi = emb0_v[pl.ds(16, 16)]
    lane = lax.iota(jnp.int32, 16)
    tail_mask = lane >= 8
    zero_v = jnp.zeros((16,), jnp.float32)
    hist_v = jnp.full((16,), float(HIST), jnp.float32)

    def issue(r, s):
        pltpu.async_copy(emb_hbm.at[idx_v.at[r, pl.ds(0, GA)]],
                         rows_a.at[pl.ds(s * GA, GA)], sems.at[s])
        pltpu.async_copy(emb_hbm.at[idx_v.at[r, pl.ds(GA, GB)]],
                         rows_b.at[pl.ds(s * GB, GB)], sems.at[s])

    def drain(s):
        pltpu.make_async_copy(emb_hbm.at[pl.ds(0, GA)],
                              rows_a.at[pl.ds(s * GA, GA)], sems.at[s]).wait()
        pltpu.make_async_copy(emb_hbm.at[pl.ds(0, GB)],
                              rows_b.at[pl.ds(s * GB, GB)], sems.at[s]).wait()

    def chunk_body(c, _):
        base = row0 + c * CHUNK
        pltpu.sync_copy(idx_hbm.at[pl.ds(base, CHUNK)], idx_v)
        for s in range(NBUF - 1):
            issue(s, s)

        def block_body(q, _):
            r0 = q * NBUF
            for s in range(NBUF):
                r = r0 + s
                nxt = r + (NBUF - 1)

                @pl.when(nxt < CHUNK)
                def _():
                    issue(nxt, (s + NBUF - 1) % NBUF)

                cnt = jnp.zeros((16,), jnp.int32)
                for k in range(12):
                    iszero = idx_v[r, pl.ds(k * 16, 16)] == 0
                    cnt = cnt + plsc.all_reduce_population_count(iszero)
                tail_zero = jnp.logical_and(idx_v[r, pl.ds(HIST - 16, 16)] == 0,
                                            tail_mask)
                cnt = cnt + plsc.all_reduce_population_count(tail_zero)
                nzv = cnt.astype(jnp.float32)

                drain(s)

                def acc_a(t, carry):
                    lo, hi = carry
                    j0 = s * GA + t * 8
                    for u in range(8):
                        lo = lo + rows_a[j0 + u, pl.ds(0, 16)]
                        hi = hi + rows_a[j0 + u, pl.ds(16, 16)]
                    return lo, hi

                lo, hi = lax.fori_loop(0, GA // 8, acc_a, (zero_v, zero_v))

                def acc_b(t, carry):
                    lo, hi = carry
                    j0 = s * GB + t * 8
                    for u in range(8):
                        lo = lo + rows_b[j0 + u, pl.ds(0, 16)]
                        hi = hi + rows_b[j0 + u, pl.ds(16, 16)]
                    return lo, hi

                lo, hi = lax.fori_loop(0, GB // 8, acc_b, (lo, hi))
                denom = (hist_v - nzv) + 1e-8
                out_v[r, pl.ds(0, 16)] = (lo - nzv * e0_lo) / denom
                out_v[r, pl.ds(16, 16)] = (hi - nzv * e0_hi) / denom
            return 0

        lax.fori_loop(0, CHUNK // NBUF, block_body, 0)
        pltpu.sync_copy(out_v, out_hbm.at[pl.ds(base, CHUNK)])
        return 0

    lax.fori_loop(0, NCHUNK, chunk_body, 0)


def kernel(inputs, embeddings):
    mesh = plsc.VectorSubcoreMesh(core_axis_name="c", subcore_axis_name="s",
                                  num_cores=NC, num_subcores=NS)
    f = pl.kernel(
        _sc_body,
        out_type=jax.ShapeDtypeStruct((B, D), jnp.float32),
        mesh=mesh,
        compiler_params=pltpu.CompilerParams(needs_layout_passes=False,
                                             use_tc_tiling_on_sc=False),
        scratch_types=[
            pltpu.VMEM((CHUNK, HIST), jnp.int32),
            pltpu.VMEM((NBUF * GA, D), jnp.float32),
            pltpu.VMEM((NBUF * GB, D), jnp.float32),
            pltpu.VMEM((D,), jnp.float32),
            pltpu.VMEM((CHUNK, D), jnp.float32),
            pltpu.SemaphoreType.DMA((NBUF,)),
        ],
    )
    return f(inputs, embeddings)


# speedup vs baseline: 16.3733x; 1.0019x over previous
import jax
import jax.numpy as jnp
from jax import lax
from jax.experimental import pallas as pl
from jax.experimental.pallas import tpu as pltpu
from jax.experimental.pallas import tpu_sc as plsc

B = 16384
HIST = 200
D = 32
NC = 2
NS = 16
NW = NC * NS
RPW = B // NW
CHUNK = 64
NCHUNK = RPW // CHUNK
GA = 128
GB = HIST - GA
NBUF = 4


def _detile_body(in_hbm, outa_hbm, outb_hbm, va, vb, loadsems, storesem):
    wid = lax.axis_index("s") * NC + lax.axis_index("c")
    row0 = wid * RPW

    colb = pl.multiple_of(GA + row0 * 0, GA)

    def load(c, p):
        base = row0 + c * CHUNK
        pltpu.async_copy(in_hbm.at[pl.ds(base, CHUNK), pl.ds(0, GA)],
                         va.at[pl.ds(p * CHUNK, CHUNK)], loadsems.at[p])
        pltpu.async_copy(in_hbm.at[pl.ds(base, CHUNK), pl.ds(colb, GA)],
                         vb.at[pl.ds(p * CHUNK, CHUNK)], loadsems.at[p])

    def wait_load(p):
        pltpu.make_async_copy(in_hbm.at[pl.ds(0, CHUNK), pl.ds(0, GA)],
                              va.at[pl.ds(p * CHUNK, CHUNK)],
                              loadsems.at[p]).wait()
        pltpu.make_async_copy(in_hbm.at[pl.ds(0, CHUNK), pl.ds(0, GA)],
                              vb.at[pl.ds(p * CHUNK, CHUNK)],
                              loadsems.at[p]).wait()

    def store(c, p):
        base = row0 + c * CHUNK
        pltpu.async_copy(va.at[pl.ds(p * CHUNK, CHUNK)],
                         outa_hbm.at[pl.ds(base, CHUNK)], storesem)
        pltpu.async_copy(vb.at[pl.ds(p * CHUNK, CHUNK)],
                         outb_hbm.at[pl.ds(base, CHUNK)], storesem)

    def wait_store():
        pltpu.make_async_copy(va.at[pl.ds(0, CHUNK)],
                              outa_hbm.at[pl.ds(0, CHUNK)], storesem).wait()
        pltpu.make_async_copy(vb.at[pl.ds(0, CHUNK)],
                              outb_hbm.at[pl.ds(0, CHUNK)], storesem).wait()

    load(0, 0)
    for c in range(NCHUNK):
        p = c % 2
        if c >= 1:
            wait_store()
        if c + 1 < NCHUNK:
            load(c + 1, 1 - p)
        wait_load(p)
        store(c, p)
    wait_store()


def _gather_body(idxa_hbm, idxb_hbm, emb_hbm, out_hbm,
                 idxa_v, idxb_v, rows_a, rows_b, emb0_v, out_v, sems):
    wid = lax.axis_index("s") * NC + lax.axis_index("c")
    row0 = wid * RPW
    pltpu.sync_copy(emb_hbm.at[0], emb0_v)
    e0_lo = emb0_v[pl.ds(0, 16)]
    e0_hi = emb0_v[pl.ds(16, 16)]
    lane = lax.iota(jnp.int32, 16)
    tail_mask = lane >= 8
    zero_v = jnp.zeros((16,), jnp.float32)
    hist_v = jnp.full((16,), float(HIST), jnp.float32)

    def issue(r, s):
        pltpu.async_copy(emb_hbm.at[idxa_v.at[r]],
                         rows_a.at[pl.ds(s * GA, GA)], sems.at[s])
        pltpu.async_copy(emb_hbm.at[idxb_v.at[r, pl.ds(0, GB)]],
                         rows_b.at[pl.ds(s * GB, GB)], sems.at[s])

    def drain(s):
        pltpu.make_async_copy(emb_hbm.at[pl.ds(0, GA)],
                              rows_a.at[pl.ds(s * GA, GA)], sems.at[s]).wait()
        pltpu.make_async_copy(emb_hbm.at[pl.ds(0, GB)],
                              rows_b.at[pl.ds(s * GB, GB)], sems.at[s]).wait()

    def chunk_body(c, _):
        base = row0 + c * CHUNK
        pltpu.sync_copy(idxa_hbm.at[pl.ds(base, CHUNK)], idxa_v)
        pltpu.sync_copy(idxb_hbm.at[pl.ds(base, CHUNK)], idxb_v)
        for s in range(NBUF - 1):
            issue(s, s)

        def block_body(q, _):
            r0 = q * NBUF
            for s in range(NBUF):
                r = r0 + s
                nxt = r + (NBUF - 1)

                @pl.when(nxt < CHUNK)
                def _():
                    issue(nxt, (s + NBUF - 1) % NBUF)

                cnt = jnp.zeros((16,), jnp.int32)
                for k in range(8):
                    iszero = idxa_v[r, pl.ds(k * 16, 16)] == 0
                    cnt = cnt + plsc.all_reduce_population_count(iszero)
                for k in range(4):
                    iszero = idxb_v[r, pl.ds(k * 16, 16)] == 0
                    cnt = cnt + plsc.all_reduce_population_count(iszero)
                tail_zero = jnp.logical_and(idxb_v[r, pl.ds(GB - 16, 16)] == 0,
                                            tail_mask)
                cnt = cnt + plsc.all_reduce_population_count(tail_zero)
                nzv = cnt.astype(jnp.float32)

                drain(s)

                def acc_a(t, carry):
                    lo, hi = carry
                    j0 = s * GA + t * 8
                    for u in range(8):
                        lo = lo + rows_a[j0 + u, pl.ds(0, 16)]
                        hi = hi + rows_a[j0 + u, pl.ds(16, 16)]
                    return lo, hi

                lo, hi = lax.fori_loop(0, GA // 8, acc_a, (zero_v, zero_v))

                def acc_b(t, carry):
                    lo, hi = carry
                    j0 = s * GB + t * 8
                    for u in range(8):
                        lo = lo + rows_b[j0 + u, pl.ds(0, 16)]
                        hi = hi + rows_b[j0 + u, pl.ds(16, 16)]
                    return lo, hi

                lo, hi = lax.fori_loop(0, GB // 8, acc_b, (lo, hi))
                denom = (hist_v - nzv) + 1e-8
                out_v[r, pl.ds(0, 16)] = (lo - nzv * e0_lo) / denom
                out_v[r, pl.ds(16, 16)] = (hi - nzv * e0_hi) / denom
            return 0

        lax.fori_loop(0, CHUNK // NBUF, block_body, 0)
        pltpu.sync_copy(out_v, out_hbm.at[pl.ds(base, CHUNK)])
        return 0

    lax.fori_loop(0, NCHUNK, chunk_body, 0)


def kernel(inputs, embeddings):
    mesh = plsc.VectorSubcoreMesh(core_axis_name="c", subcore_axis_name="s",
                                  num_cores=NC, num_subcores=NS)
    detile = pl.kernel(
        _detile_body,
        out_type=(jax.ShapeDtypeStruct((B, GA), jnp.int32),
                  jax.ShapeDtypeStruct((B, GA), jnp.int32)),
        mesh=mesh,
        compiler_params=pltpu.CompilerParams(needs_layout_passes=False,
                                             use_tc_tiling_on_sc=True),
        scratch_types=[
            pltpu.VMEM((2 * CHUNK, GA), jnp.int32),
            pltpu.VMEM((2 * CHUNK, GA), jnp.int32),
            pltpu.SemaphoreType.DMA((2,)),
            pltpu.SemaphoreType.DMA,
        ],
    )
    gather = pl.kernel(
        _gather_body,
        out_type=jax.ShapeDtypeStruct((B, D), jnp.float32),
        mesh=mesh,
        compiler_params=pltpu.CompilerParams(needs_layout_passes=False,
                                             use_tc_tiling_on_sc=False),
        scratch_types=[
            pltpu.VMEM((CHUNK, GA), jnp.int32),
            pltpu.VMEM((CHUNK, GA), jnp.int32),
            pltpu.VMEM((NBUF * GA, D), jnp.float32),
            pltpu.VMEM((NBUF * GB, D), jnp.float32),
            pltpu.VMEM((D,), jnp.float32),
            pltpu.VMEM((CHUNK, D), jnp.float32),
            pltpu.SemaphoreType.DMA((NBUF,)),
        ],
    )
    idxa, idxb = detile(inputs)
    return gather(idxa, idxb, embeddings)
